# Initial kernel scaffold; baseline (speedup 1.0000x reference)
import jax
import jax.numpy as jnp
from jax.experimental import pallas as pl


def kernel(recon_grf, original_grf):
    raise NotImplementedError("write your pallas kernel here")


# speedup vs baseline: 23.7702x; 23.7702x over previous
import jax
import jax.numpy as jnp
from jax.experimental import pallas as pl

_N = 64
_NUM_TRIU = _N * (_N + 1) // 2


def _bce_triu_kernel(p_ref, t_ref, o_ref):
    p = p_ref[...]
    t = t_ref[...]
    log_p = jnp.maximum(jnp.log(p), -100.0)
    log_1p = jnp.maximum(jnp.log(1.0 - p), -100.0)
    term = t * log_p + (1.0 - t) * log_1p
    rows = jax.lax.broadcasted_iota(jnp.int32, (_N, _N), 0)
    cols = jax.lax.broadcasted_iota(jnp.int32, (_N, _N), 1)
    mask = rows <= cols
    total = jnp.sum(jnp.where(mask, term, 0.0), keepdims=True)
    o_ref[...] = -total / _NUM_TRIU


def kernel(recon_grf, original_grf):
    out = pl.pallas_call(
        _bce_triu_kernel,
        out_shape=jax.ShapeDtypeStruct((1, 1), jnp.float32),
    )(recon_grf, original_grf)
    return out[0, 0]
